# baseline (speedup 1.0000x reference)
import functools

import jax
import jax.numpy as jnp
from jax import lax
from jax.experimental import pallas as pl
from jax.experimental.pallas import tpu as pltpu
from jax.experimental.pallas import tpu_sc as plsc

HIDDEN_DIM = 512
BATCH = 16384
LANES = 16
JV = HIDDEN_DIM // LANES

_info = plsc.get_sparse_core_info()
NC, NS = _info.num_cores, _info.num_subcores
NW = NC * NS
B_PER_W = BATCH // NW
CHUNK = 64
N_CHUNKS = B_PER_W // CHUNK
NGRP = B_PER_W // LANES
LIST_PAD = B_PER_W + LANES


def _perm(x, idx):
    return lax.gather(
        x, idx.reshape(LANES, 1),
        lax.GatherDimensionNumbers(
            offset_dims=(), collapsed_slice_dims=(0,), start_index_map=(0,)),
        (1,), mode=lax.GatherScatterMode.PROMISE_IN_BOUNDS)


def _mesh_kernel():
    mesh = plsc.VectorSubcoreMesh(core_axis_name="c", subcore_axis_name="s")

    @functools.partial(
        pl.kernel,
        mesh=mesh,
        out_type=jax.ShapeDtypeStruct((BATCH, HIDDEN_DIM), jnp.float32),
        scratch_types=[
            pltpu.VMEM((B_PER_W,), jnp.int32),
            pltpu.VMEM((2, HIDDEN_DIM), jnp.float32),
            pltpu.VMEM((CHUNK, HIDDEN_DIM), jnp.float32),
            pltpu.VMEM((CHUNK, HIDDEN_DIM), jnp.float32),
            pltpu.VMEM((LIST_PAD,), jnp.int32),
            pltpu.VMEM((LIST_PAD,), jnp.int32),
            pltpu.VMEM((N_CHUNKS, CHUNK), jnp.int32),
            pltpu.VMEM((N_CHUNKS, CHUNK), jnp.int32),
        ],
    )
    def body(table_hbm, idx_hbm, out_hbm, idx_v, tab_v, buf0, buf1,
             zl, ol, z2, o2):
        wid = lax.axis_index("s") * NC + lax.axis_index("c")
        base = wid * B_PER_W
        pltpu.sync_copy(idx_hbm.at[wid], idx_v)
        pltpu.sync_copy(table_hbm, tab_v)

        lane = lax.iota(jnp.int32, LANES)

        w0 = [tab_v[0, pl.ds(j * LANES, LANES)] for j in range(JV)]
        w1 = [tab_v[1, pl.ds(j * LANES, LANES)] for j in range(JV)]

        def fill_body(i, _):
            for j in range(JV):
                buf0[i, pl.ds(j * LANES, LANES)] = w0[j]
                buf1[i, pl.ds(j * LANES, LANES)] = w1[j]
            return 0

        lax.fori_loop(0, CHUNK, fill_body, 0)

        def part_body(t, carry):
            zoff, ooff = carry
            v = idx_v[pl.ds(t * LANES, LANES)]
            pos = (base + t * LANES) + lane
            mz = v == 0
            x = jnp.where(mz, 1, 0)
            for sh in (1, 2, 4, 8):
                shifted = _perm(x, jnp.maximum(lane - sh, 0))
                x = x + jnp.where(lane >= sh, shifted, 0)
            nz = x[LANES - 1]
            dest = jnp.where(mz, x - 1, (nz + lane) - x)
            src = jnp.zeros((LANES,), jnp.int32)
            for j in range(LANES):
                src = jnp.where(lane == dest[j], j, src)
            spos = _perm(pos, src)
            zl[pl.ds(zoff, LANES)] = spos
            ol[pl.ds(ooff, LANES)] = lax.rev(spos, (0,))
            return zoff + nz, ooff + (LANES - nz)

        zcnt, ocnt = lax.fori_loop(0, NGRP, part_body,
                                   (jnp.int32(0), jnp.int32(0)))

        def pad_list(lst, cnt):
            last = jnp.maximum(cnt - 1, 0)
            w = lst[pl.ds(last, LANES)]
            lastv = lax.broadcast_in_dim(w[0], (LANES,), ())
            for t in range(NGRP):
                at = t * LANES + lane
                cur = lst[pl.ds(t * LANES, LANES)]
                lst[pl.ds(t * LANES, LANES)] = jnp.where(at <= last, cur,
                                                         lastv)

        pad_list(zl, zcnt)
        pad_list(ol, ocnt)

        for t in range(N_CHUNKS):
            for u in range(CHUNK // LANES):
                z2[t, pl.ds(u * LANES, LANES)] = zl[
                    pl.ds(t * CHUNK + u * LANES, LANES)]
                o2[t, pl.ds(u * LANES, LANES)] = ol[
                    pl.ds(t * CHUNK + u * LANES, LANES)]

        for c in range(N_CHUNKS):
            @pl.when(zcnt > c * CHUNK)
            def _():
                pltpu.sync_copy(buf0, out_hbm.at[z2.at[c]])

            @pl.when(ocnt > c * CHUNK)
            def _():
                pltpu.sync_copy(buf1, out_hbm.at[o2.at[c]])

    return body


_sc_lookup = _mesh_kernel()


@jax.jit
def kernel(domain_ids, embed_weight):
    ids = domain_ids.astype(jnp.int32).reshape(NW, B_PER_W)
    return _sc_lookup(embed_weight, ids)


# speedup vs baseline: 1.0034x; 1.0034x over previous
import functools

import jax
import jax.numpy as jnp
from jax import lax
from jax.experimental import pallas as pl
from jax.experimental.pallas import tpu as pltpu
from jax.experimental.pallas import tpu_sc as plsc

HIDDEN_DIM = 512
BATCH = 16384
LANES = 16
JV = HIDDEN_DIM // LANES

_info = plsc.get_sparse_core_info()
NC, NS = _info.num_cores, _info.num_subcores
NW = NC * NS
B_PER_W = BATCH // NW
CHUNK = 64
N_CHUNKS = B_PER_W // CHUNK
NGRP = B_PER_W // LANES
LIST_PAD = B_PER_W + LANES


def _perm(x, idx):
    return lax.gather(
        x, idx.reshape(LANES, 1),
        lax.GatherDimensionNumbers(
            offset_dims=(), collapsed_slice_dims=(0,), start_index_map=(0,)),
        (1,), mode=lax.GatherScatterMode.PROMISE_IN_BOUNDS)


def _mesh_kernel():
    mesh = plsc.VectorSubcoreMesh(core_axis_name="c", subcore_axis_name="s")

    @functools.partial(
        pl.kernel,
        mesh=mesh,
        out_type=jax.ShapeDtypeStruct((BATCH, HIDDEN_DIM), jnp.float32),
        scratch_types=[
            pltpu.VMEM((B_PER_W,), jnp.int32),
            pltpu.VMEM((2, HIDDEN_DIM), jnp.float32),
            pltpu.VMEM((CHUNK, HIDDEN_DIM), jnp.float32),
            pltpu.VMEM((CHUNK, HIDDEN_DIM), jnp.float32),
            pltpu.VMEM((LIST_PAD,), jnp.int32),
            pltpu.VMEM((LIST_PAD,), jnp.int32),
            pltpu.VMEM((N_CHUNKS, CHUNK), jnp.int32),
            pltpu.VMEM((N_CHUNKS, CHUNK), jnp.int32),
            pltpu.SemaphoreType.DMA,
            pltpu.SemaphoreType.DMA,
        ],
    )
    def body(table_hbm, idx_hbm, out_hbm, idx_v, tab_v, buf0, buf1,
             zl, ol, z2, o2, semz, semo):
        wid = lax.axis_index("s") * NC + lax.axis_index("c")
        base = wid * B_PER_W
        pltpu.sync_copy(idx_hbm.at[wid], idx_v)
        pltpu.sync_copy(table_hbm, tab_v)

        lane = lax.iota(jnp.int32, LANES)

        w0 = [tab_v[0, pl.ds(j * LANES, LANES)] for j in range(JV)]
        w1 = [tab_v[1, pl.ds(j * LANES, LANES)] for j in range(JV)]

        def fill_body(i, _):
            for j in range(JV):
                buf0[i, pl.ds(j * LANES, LANES)] = w0[j]
                buf1[i, pl.ds(j * LANES, LANES)] = w1[j]
            return 0

        lax.fori_loop(0, CHUNK, fill_body, 0)

        def part_body(t, carry):
            zoff, ooff = carry
            v = idx_v[pl.ds(t * LANES, LANES)]
            pos = (base + t * LANES) + lane
            mz = v == 0
            x = jnp.where(mz, 1, 0)
            for sh in (1, 2, 4, 8):
                shifted = _perm(x, jnp.maximum(lane - sh, 0))
                x = x + jnp.where(lane >= sh, shifted, 0)
            nz = x[LANES - 1]
            dest = jnp.where(mz, x - 1, (nz + lane) - x)
            src = jnp.zeros((LANES,), jnp.int32)
            for j in range(LANES):
                src = jnp.where(lane == dest[j], j, src)
            spos = _perm(pos, src)
            zl[pl.ds(zoff, LANES)] = spos
            ol[pl.ds(ooff, LANES)] = lax.rev(spos, (0,))
            return zoff + nz, ooff + (LANES - nz)

        zcnt, ocnt = lax.fori_loop(0, NGRP, part_body,
                                   (jnp.int32(0), jnp.int32(0)))

        def pad_list(lst, cnt):
            last = jnp.maximum(cnt - 1, 0)
            w = lst[pl.ds(last, LANES)]
            lastv = lax.broadcast_in_dim(w[0], (LANES,), ())
            for t in range(NGRP):
                at = t * LANES + lane
                cur = lst[pl.ds(t * LANES, LANES)]
                lst[pl.ds(t * LANES, LANES)] = jnp.where(at <= last, cur,
                                                         lastv)

        pad_list(zl, zcnt)
        pad_list(ol, ocnt)

        for t in range(N_CHUNKS):
            for u in range(CHUNK // LANES):
                z2[t, pl.ds(u * LANES, LANES)] = zl[
                    pl.ds(t * CHUNK + u * LANES, LANES)]
                o2[t, pl.ds(u * LANES, LANES)] = ol[
                    pl.ds(t * CHUNK + u * LANES, LANES)]

        for c in range(N_CHUNKS):
            @pl.when(zcnt > c * CHUNK)
            def _():
                pltpu.async_copy(buf0, out_hbm.at[z2.at[c]], semz)

            @pl.when(ocnt > c * CHUNK)
            def _():
                pltpu.async_copy(buf1, out_hbm.at[o2.at[c]], semo)

        for c in range(N_CHUNKS):
            @pl.when(zcnt > c * CHUNK)
            def _():
                pltpu.make_async_copy(buf0, out_hbm.at[z2.at[c]], semz).wait()

            @pl.when(ocnt > c * CHUNK)
            def _():
                pltpu.make_async_copy(buf1, out_hbm.at[o2.at[c]], semo).wait()

    return body


_sc_lookup = _mesh_kernel()


@jax.jit
def kernel(domain_ids, embed_weight):
    ids = domain_ids.astype(jnp.int32).reshape(NW, B_PER_W)
    return _sc_lookup(embed_weight, ids)
